# baseline (speedup 1.0000x reference)
import jax
import jax.numpy as jnp
from jax import lax
from jax.experimental import pallas as pl
from jax.experimental.pallas import tpu as pltpu
from jax.experimental.pallas import tpu_sc as plsc

B = 32768
D = 128
DW = D // 2
MAX_POS = 32768
NC = 2
NS = 16
NW = NC * NS
BPW = B // NW
CHUNK = 128
NCHUNK = BPW // CHUNK
PAIRS = CHUNK // 2


def _gather_body(pids_hbm, cos_hbm, sin_hbm, cos_out, sin_out,
                 idx_v, ipair_v, cos_b0, cos_b1, sin_b0, sin_b1,
                 cos_e0, cos_e1, sin_e0, sin_e1, sems, wsems):
    wid = lax.axis_index("s") * NC + lax.axis_index("c")
    pair_base = wid * (BPW // 2)
    cos_t = cos_hbm.bitcast(jnp.int32)
    sin_t = sin_hbm.bitcast(jnp.int32)
    cos_o = cos_out.bitcast(jnp.int32)
    sin_o = sin_out.bitcast(jnp.int32)

    pltpu.sync_copy(pids_hbm.at[wid], idx_v)

    def ipair_compute(c):
        for v in range(CHUNK // 16):
            s = pl.ds(v * 16, 16)
            ipair_v[c, s] = lax.shift_right_logical(idx_v[c, s], 1)

    cbufs = (cos_b0, cos_b1)
    sbufs = (sin_b0, sin_b1)

    def fire(c, slot):
        return (pltpu.async_copy(cos_t.at[ipair_v.at[c]], cbufs[slot],
                                 sems.at[2 * slot]),
                pltpu.async_copy(sin_t.at[ipair_v.at[c]], sbufs[slot],
                                 sems.at[2 * slot + 1]))

    ipair_compute(0)
    inflight = fire(0, 0)
    for c in range(1, NCHUNK):
        ipair_compute(c)
    wb = [None, None]
    for c in range(NCHUNK):
        slot = c % 2
        nxt = None
        if c + 1 < NCHUNK:
            nxt = fire(c + 1, 1 - slot)
        for cp in inflight:
            cp.wait()
        inflight = nxt
        cbuf = cbufs[slot]
        sbuf = sbufs[slot]
        cos_ext = (cos_e0, cos_e1)[slot]
        sin_ext = (sin_e0, sin_e1)[slot]
        if c >= 2:
            for cp in wb[slot]:
                cp.wait()

        @plsc.parallel_loop(0, CHUNK // 16)
        def ext(g):
            vec = idx_v[c, pl.ds(g * 16, 16)]
            sh = (vec & 1) * 16
            svs = [pl.ds(v * 16, 16) for v in range(D // 16)]

            def loads(buf, u):
                j0 = g * 16 + 2 * u
                return ([buf[j0, sv] for sv in svs],
                        [buf[j0 + 1, sv] for sv in svs])

            for buf, ext_ref in ((cbuf, cos_ext), (sbuf, sin_ext)):
                w0, w1 = loads(buf, 0)
                for u in range(8):
                    nxt_w = loads(buf, u + 1) if u + 1 < 8 else None
                    s0 = sh[2 * u]
                    s1 = sh[2 * u + 1]
                    k = (g * 16 - 2 * u + 2 * u) // 2 + u
                    for v, sv in enumerate(svs):
                        lo = lax.shift_right_logical(w0[v], s0) & 0xFFFF
                        hi = lax.shift_left(lax.shift_right_logical(w1[v], s1), 16)
                        ext_ref[k, sv] = lo | hi
                    if nxt_w is not None:
                        w0, w1 = nxt_w

        off = pair_base + c * PAIRS
        wb[slot] = (
            pltpu.async_copy(cos_ext, cos_o.at[pl.ds(off, PAIRS)],
                             wsems.at[2 * slot]),
            pltpu.async_copy(sin_ext, sin_o.at[pl.ds(off, PAIRS)],
                             wsems.at[2 * slot + 1]),
        )
    for s in range(2):
        for cp in wb[s]:
            cp.wait()


def kernel(position_ids, cos_cached, sin_cached):
    bsz, seqlen = position_ids.shape
    pids = position_ids.reshape(NW, NCHUNK, CHUNK).astype(jnp.int32)
    out_sds = jax.ShapeDtypeStruct((B, D), cos_cached.dtype)
    kfn = pl.kernel(
        _gather_body,
        out_type=[out_sds, out_sds],
        mesh=plsc.VectorSubcoreMesh(core_axis_name="c", subcore_axis_name="s"),
        scratch_types=[
            pltpu.VMEM((NCHUNK, CHUNK), jnp.int32),
            pltpu.VMEM((NCHUNK, CHUNK), jnp.int32),
            pltpu.VMEM((CHUNK, D), jnp.int32),
            pltpu.VMEM((CHUNK, D), jnp.int32),
            pltpu.VMEM((CHUNK, D), jnp.int32),
            pltpu.VMEM((CHUNK, D), jnp.int32),
            pltpu.VMEM((PAIRS, D), jnp.int32),
            pltpu.VMEM((PAIRS, D), jnp.int32),
            pltpu.VMEM((PAIRS, D), jnp.int32),
            pltpu.VMEM((PAIRS, D), jnp.int32),
            pltpu.SemaphoreType.DMA((4,)),
            pltpu.SemaphoreType.DMA((4,)),
        ],
    )
    cos_flat, sin_flat = kfn(pids, cos_cached, sin_cached)
    shape = (bsz, seqlen, D)
    return cos_flat.reshape(shape), sin_flat.reshape(shape)


# speedup vs baseline: 1.0525x; 1.0525x over previous
import jax
import jax.numpy as jnp
from jax import lax
from jax.experimental import pallas as pl
from jax.experimental.pallas import tpu as pltpu
from jax.experimental.pallas import tpu_sc as plsc

B = 32768
D = 128
DW = D // 2
MAX_POS = 32768
NC = 2
NS = 16
NW = NC * NS
BPW = B // NW
CHUNK = 128
NCHUNK = BPW // CHUNK
PAIRS = CHUNK // 2


def _gather_body(pids_hbm, cos_hbm, sin_hbm, cos_out, sin_out,
                 idx_v, ipair_v, cos_b0, cos_b1, sin_b0, sin_b1,
                 cos_e0, cos_e1, sin_e0, sin_e1, sems, wsems):
    wid = lax.axis_index("s") * NC + lax.axis_index("c")
    pair_base = wid * (BPW // 2)
    cos_t = cos_hbm.bitcast(jnp.int32)
    sin_t = sin_hbm.bitcast(jnp.int32)
    cos_o = cos_out.bitcast(jnp.int32)
    sin_o = sin_out.bitcast(jnp.int32)

    pltpu.sync_copy(pids_hbm.at[wid], idx_v)

    def ipair_compute(c):
        for v in range(CHUNK // 16):
            s = pl.ds(v * 16, 16)
            ipair_v[c, s] = lax.shift_right_logical(idx_v[c, s], 1)

    cbufs = (cos_b0, cos_b1)
    sbufs = (sin_b0, sin_b1)

    def fire(c, slot):
        return (pltpu.async_copy(cos_t.at[ipair_v.at[c]], cbufs[slot],
                                 sems.at[2 * slot]),
                pltpu.async_copy(sin_t.at[ipair_v.at[c]], sbufs[slot],
                                 sems.at[2 * slot + 1]))

    ipair_compute(0)
    inflight = fire(0, 0)
    for c in range(1, NCHUNK):
        ipair_compute(c)
    wb = [None, None]
    for c in range(NCHUNK):
        slot = c % 2
        nxt = None
        if c + 1 < NCHUNK:
            nxt = fire(c + 1, 1 - slot)
        for cp in inflight:
            cp.wait()
        inflight = nxt
        cbuf = cbufs[slot]
        sbuf = sbufs[slot]
        cos_ext = (cos_e0, cos_e1)[slot]
        sin_ext = (sin_e0, sin_e1)[slot]
        if c >= 2:
            for cp in wb[slot]:
                cp.wait()

        @plsc.parallel_loop(0, CHUNK // 16)
        def ext(g):
            vec = idx_v[c, pl.ds(g * 16, 16)]
            sh = (vec & 1) * 16
            svs = [pl.ds(v * 16, 16) for v in range(D // 16)]

            def loads(buf, u):
                j0 = g * 16 + 2 * u
                return ([buf[j0, sv] for sv in svs],
                        [buf[j0 + 1, sv] for sv in svs])

            for buf, ext_ref in ((cbuf, cos_ext), (sbuf, sin_ext)):
                w0, w1 = loads(buf, 0)
                for u in range(8):
                    s0 = sh[2 * u]
                    s1 = sh[2 * u + 1]
                    k = (g * 16 - 2 * u + 2 * u) // 2 + u
                    nw0, nw1 = [], []
                    for v, sv in enumerate(svs):
                        if u + 1 < 8:
                            j0n = g * 16 + 2 * (u + 1)
                            nw0.append(buf[j0n, sv])
                            nw1.append(buf[j0n + 1, sv])
                        lo = lax.shift_right_logical(w0[v], s0) & 0xFFFF
                        hi = lax.shift_left(lax.shift_right_logical(w1[v], s1), 16)
                        ext_ref[k, sv] = lo | hi
                    if nw0:
                        w0, w1 = nw0, nw1

        off = pair_base + c * PAIRS
        wb[slot] = (
            pltpu.async_copy(cos_ext, cos_o.at[pl.ds(off, PAIRS)],
                             wsems.at[2 * slot]),
            pltpu.async_copy(sin_ext, sin_o.at[pl.ds(off, PAIRS)],
                             wsems.at[2 * slot + 1]),
        )
    for s in range(2):
        for cp in wb[s]:
            cp.wait()


def kernel(position_ids, cos_cached, sin_cached):
    bsz, seqlen = position_ids.shape
    pids = position_ids.reshape(NW, NCHUNK, CHUNK).astype(jnp.int32)
    out_sds = jax.ShapeDtypeStruct((B, D), cos_cached.dtype)
    kfn = pl.kernel(
        _gather_body,
        out_type=[out_sds, out_sds],
        mesh=plsc.VectorSubcoreMesh(core_axis_name="c", subcore_axis_name="s"),
        scratch_types=[
            pltpu.VMEM((NCHUNK, CHUNK), jnp.int32),
            pltpu.VMEM((NCHUNK, CHUNK), jnp.int32),
            pltpu.VMEM((CHUNK, D), jnp.int32),
            pltpu.VMEM((CHUNK, D), jnp.int32),
            pltpu.VMEM((CHUNK, D), jnp.int32),
            pltpu.VMEM((CHUNK, D), jnp.int32),
            pltpu.VMEM((PAIRS, D), jnp.int32),
            pltpu.VMEM((PAIRS, D), jnp.int32),
            pltpu.VMEM((PAIRS, D), jnp.int32),
            pltpu.VMEM((PAIRS, D), jnp.int32),
            pltpu.SemaphoreType.DMA((4,)),
            pltpu.SemaphoreType.DMA((4,)),
        ],
    )
    cos_flat, sin_flat = kfn(pids, cos_cached, sin_cached)
    shape = (bsz, seqlen, D)
    return cos_flat.reshape(shape), sin_flat.reshape(shape)
